# baseline (speedup 1.0000x reference)
import functools

import jax
import jax.numpy as jnp
from jax import lax
from jax.experimental import pallas as pl
from jax.experimental.pallas import tpu as pltpu

HIDDEN_DIM = 2048
NUM_EXPERTS = 16
TOP_K = 2
N_TOKENS = 16384

CHUNK_T = 256
NBUF = 12
NCHUNK = N_TOKENS // CHUNK_T
TAIL_T = 1024
NTAIL = N_TOKENS // TAIL_T


def _router_body(x_hbm, w_ref, b_ref,
                 logits_ref, weights_ref, idx_ref, mask_ref,
                 xbuf, lt_ref, wr_ref, ir_ref, sems):
    w = w_ref[...]
    b = b_ref[...]

    def chunk_copy(c):
        slot = c % NBUF
        return pltpu.make_async_copy(
            x_hbm.at[pl.ds(c * CHUNK_T, CHUNK_T), :],
            xbuf.at[slot],
            sems.at[slot],
        )

    for c in range(NBUF - 1):
        chunk_copy(c).start()

    iota_e = lax.broadcasted_iota(jnp.int32, (NUM_EXPERTS, CHUNK_T), 0)

    for c in range(NCHUNK):
        nxt = c + NBUF - 1
        if nxt < NCHUNK:
            chunk_copy(nxt).start()
        chunk_copy(c).wait()
        x = xbuf[c % NBUF]

        logits_t = lax.dot_general(
            w, x, dimension_numbers=(((1,), (1,)), ((), ())),
            preferred_element_type=jnp.float32,
        ) + b
        tok = pl.ds(c * CHUNK_T, CHUNK_T)
        lt_ref[:, tok] = logits_t

        m1 = jnp.max(logits_t, axis=0, keepdims=True)
        i1 = jnp.min(jnp.where(logits_t == m1, iota_e, NUM_EXPERTS),
                     axis=0, keepdims=True)
        masked = jnp.where(iota_e == i1, -jnp.inf, logits_t)
        m2 = jnp.max(masked, axis=0, keepdims=True)
        i2 = jnp.min(jnp.where(masked == m2, iota_e, NUM_EXPERTS),
                     axis=0, keepdims=True)

        t = jnp.exp(m2 - m1)
        w1 = 1.0 / (1.0 + t)
        w2 = t * w1
        wr_ref[0:1, tok] = w1
        wr_ref[1:2, tok] = w2
        ir_ref[0:1, tok] = i1
        ir_ref[1:2, tok] = i2

        mask_ref[:, 0, tok] = (iota_e == i1).astype(jnp.int32)
        mask_ref[:, 1, tok] = (iota_e == i2).astype(jnp.int32)

    for j in range(NTAIL):
        tok = pl.ds(j * TAIL_T, TAIL_T)
        logits_ref[tok, :] = jnp.transpose(lt_ref[:, tok])
        weights_ref[tok, :] = jnp.transpose(wr_ref[:, tok])
        idx_ref[tok, :] = jnp.transpose(ir_ref[:, tok])


@functools.partial(jax.jit, static_argnames=("interpret",))
def kernel(X, W, b, interpret=False):
    n_tokens = X.shape[0]
    b2 = b.reshape(NUM_EXPERTS, 1)

    out_shapes = (
        jax.ShapeDtypeStruct((n_tokens, NUM_EXPERTS), jnp.float32),
        jax.ShapeDtypeStruct((n_tokens, TOP_K), jnp.float32),
        jax.ShapeDtypeStruct((n_tokens, TOP_K), jnp.int32),
        jax.ShapeDtypeStruct((NUM_EXPERTS, TOP_K, n_tokens), jnp.int32),
    )
    in_specs = [
        pl.BlockSpec(memory_space=pltpu.MemorySpace.HBM),
        pl.BlockSpec((NUM_EXPERTS, HIDDEN_DIM), lambda: (0, 0)),
        pl.BlockSpec((NUM_EXPERTS, 1), lambda: (0, 0)),
    ]
    out_specs = (
        pl.BlockSpec((n_tokens, NUM_EXPERTS), lambda: (0, 0)),
        pl.BlockSpec((n_tokens, TOP_K), lambda: (0, 0)),
        pl.BlockSpec((n_tokens, TOP_K), lambda: (0, 0)),
        pl.BlockSpec((NUM_EXPERTS, TOP_K, n_tokens), lambda: (0, 0, 0)),
    )
    logits, weights, idx, mask = pl.pallas_call(
        _router_body,
        in_specs=in_specs,
        out_specs=out_specs,
        out_shape=out_shapes,
        scratch_shapes=[
            pltpu.VMEM((NBUF, CHUNK_T, HIDDEN_DIM), jnp.float32),
            pltpu.VMEM((NUM_EXPERTS, N_TOKENS), jnp.float32),
            pltpu.VMEM((TOP_K, N_TOKENS), jnp.float32),
            pltpu.VMEM((TOP_K, N_TOKENS), jnp.int32),
            pltpu.SemaphoreType.DMA((NBUF,)),
        ],
        interpret=interpret,
    )(X, W, b2)
    return (logits, weights, idx, mask)


# speedup vs baseline: 1.7813x; 1.7813x over previous
import functools

import jax
import jax.numpy as jnp
from jax import lax
from jax.experimental import pallas as pl
from jax.experimental.pallas import tpu as pltpu

HIDDEN_DIM = 2048
NUM_EXPERTS = 16
TOP_K = 2
N_TOKENS = 16384

CHUNK_T = 512
NBUF = 8
NCHUNK = N_TOKENS // CHUNK_T
TAIL_T = 1024
NTAIL = N_TOKENS // TAIL_T


def _router_body(x_hbm, w_ref, b_ref,
                 logits_ref, weights_ref, idx_ref, mask_ref,
                 xbuf, sems):
    w = w_ref[...]
    b = b_ref[...]

    def chunk_copy(c):
        slot = c % NBUF
        return pltpu.make_async_copy(
            x_hbm.at[pl.ds(c * CHUNK_T, CHUNK_T), :],
            xbuf.at[slot],
            sems.at[slot],
        )

    for c in range(NBUF - 1):
        chunk_copy(c).start()

    iota_e = lax.broadcasted_iota(jnp.int32, (NUM_EXPERTS, CHUNK_T), 0)

    for c in range(NCHUNK):
        nxt = c + NBUF - 1
        if nxt < NCHUNK:
            chunk_copy(nxt).start()
        chunk_copy(c).wait()
        x = xbuf[c % NBUF]

        logits_t = lax.dot_general(
            w, x, dimension_numbers=(((1,), (1,)), ((), ())),
            preferred_element_type=jnp.float32,
        ) + b
        tok = pl.ds(c * CHUNK_T, CHUNK_T)
        logits_ref[:, tok] = logits_t

        m1 = jnp.max(logits_t, axis=0, keepdims=True)
        i1 = jnp.min(jnp.where(logits_t == m1, iota_e, NUM_EXPERTS),
                     axis=0, keepdims=True)
        masked = jnp.where(iota_e == i1, -jnp.inf, logits_t)
        m2 = jnp.max(masked, axis=0, keepdims=True)
        i2 = jnp.min(jnp.where(masked == m2, iota_e, NUM_EXPERTS),
                     axis=0, keepdims=True)

        t = jnp.exp(m2 - m1)
        w1 = 1.0 / (1.0 + t)
        w2 = t * w1
        weights_ref[0:1, tok] = w1
        weights_ref[1:2, tok] = w2
        idx_ref[0:1, tok] = i1
        idx_ref[1:2, tok] = i2

        mask_ref[:, 0, tok] = (iota_e == i1).astype(jnp.int32)
        mask_ref[:, 1, tok] = (iota_e == i2).astype(jnp.int32)


@functools.partial(jax.jit, static_argnames=("interpret",))
def kernel(X, W, b, interpret=False):
    n_tokens = X.shape[0]
    b2 = b.reshape(NUM_EXPERTS, 1)

    out_shapes = (
        jax.ShapeDtypeStruct((NUM_EXPERTS, n_tokens), jnp.float32),
        jax.ShapeDtypeStruct((TOP_K, n_tokens), jnp.float32),
        jax.ShapeDtypeStruct((TOP_K, n_tokens), jnp.int32),
        jax.ShapeDtypeStruct((NUM_EXPERTS, TOP_K, n_tokens), jnp.int32),
    )
    in_specs = [
        pl.BlockSpec(memory_space=pltpu.MemorySpace.HBM),
        pl.BlockSpec((NUM_EXPERTS, HIDDEN_DIM), lambda: (0, 0)),
        pl.BlockSpec((NUM_EXPERTS, 1), lambda: (0, 0)),
    ]
    out_specs = (
        pl.BlockSpec((NUM_EXPERTS, n_tokens), lambda: (0, 0)),
        pl.BlockSpec((TOP_K, n_tokens), lambda: (0, 0)),
        pl.BlockSpec((TOP_K, n_tokens), lambda: (0, 0)),
        pl.BlockSpec((NUM_EXPERTS, TOP_K, n_tokens), lambda: (0, 0, 0)),
    )
    logits, weights, idx, mask = pl.pallas_call(
        _router_body,
        in_specs=in_specs,
        out_specs=out_specs,
        out_shape=out_shapes,
        scratch_shapes=[
            pltpu.VMEM((NBUF, CHUNK_T, HIDDEN_DIM), jnp.float32),
            pltpu.SemaphoreType.DMA((NBUF,)),
        ],
        interpret=interpret,
    )(X, W, b2)
    return (jnp.transpose(logits), jnp.transpose(weights),
            jnp.transpose(idx), mask)
